# baseline (speedup 1.0000x reference)
import functools

import jax
import jax.numpy as jnp
from jax import lax
from jax.experimental import pallas as pl
from jax.experimental.pallas import tpu as pltpu
from jax.experimental.pallas import tpu_sc as plsc

NUM_EMB = 1000000
D = 64
BATCH = 4096
SEQ = 200
NC = 2
NS = 16
NW = NC * NS
BBLK = BATCH // NW
NBUF = 4


def _embed_body(idxt_hbm, table_hbm, out_hbm, idx_v, rows_v, tp_v, gsems, ssems):
    wid = lax.axis_index("s") * NC + lax.axis_index("c")
    pltpu.sync_copy(idxt_hbm.at[:, pl.ds(wid * BBLK, BBLK)], idx_v)

    lane = jax.lax.iota(jnp.int32, 16)

    def fire_gather(s, b):
        pltpu.async_copy(
            table_hbm.at[idx_v.at[s, pl.ds(0, BBLK)]],
            rows_v.at[b],
            gsems[b],
        )

    def wait_gather(b):
        pltpu.make_async_copy(
            table_hbm.at[pl.ds(0, BBLK)], rows_v.at[b], gsems[b]
        ).wait()

    def transpose(b):
        def cstep(c, carry):
            rbase = c * 16 + lane
            for d in range(D):
                vec = plsc.load_gather(
                    rows_v.at[b], [rbase, jnp.full((16,), d, jnp.int32)]
                )
                tp_v[b, d // 8, d % 8, pl.ds(c * 16, 16)] = vec
            return carry

        lax.fori_loop(0, BBLK // 16, cstep, 0)

    def fire_store(s, b):
        pltpu.async_copy(tp_v.at[b], out_hbm.at[s, :, wid], ssems[b])

    def wait_store_all(b):
        pltpu.make_async_copy(
            tp_v.at[b], out_hbm.at[0, :, 0], ssems[b]
        ).wait()

    for b in range(NBUF):
        fire_gather(b, b)

    def pair(g, carry):
        for b in range(NBUF):
            s = g * NBUF + b
            wait_gather(b)
            transpose(b)
            fire_store(s, b)
            wait_store_all(b)
            fire_gather(s + NBUF, b)
        return carry

    lax.fori_loop(0, SEQ // NBUF - 1, pair, 0)

    for b in range(NBUF):
        s = SEQ - NBUF + b
        wait_gather(b)
        transpose(b)
        fire_store(s, b)
    for b in range(NBUF):
        wait_store_all(b)


@jax.jit
def _embed(idxt, embedding):
    mesh = plsc.VectorSubcoreMesh(
        core_axis_name="c", subcore_axis_name="s", num_cores=NC, num_subcores=NS
    )
    return pl.kernel(
        _embed_body,
        out_type=jax.ShapeDtypeStruct((SEQ, D // 8, NW, 8, BBLK), jnp.float32),
        mesh=mesh,
        scratch_types=[
            pltpu.VMEM((SEQ, BBLK), jnp.int32),
            pltpu.VMEM((NBUF, BBLK, D), jnp.float32),
            pltpu.VMEM((NBUF, D // 8, 8, BBLK), jnp.float32),
            [pltpu.SemaphoreType.DMA] * NBUF,
            [pltpu.SemaphoreType.DMA] * NBUF,
        ],
        compiler_params=pltpu.CompilerParams(
            use_tc_tiling_on_sc=False, needs_layout_passes=False
        ),
    )(idxt, embedding)


def kernel(inputs, embedding):
    idxt = jnp.transpose(inputs)
    out5 = _embed(idxt, embedding)
    out = out5.transpose(2, 4, 0, 1, 3).reshape(BATCH, SEQ, D)
    return out


# speedup vs baseline: 1.0041x; 1.0041x over previous
import functools

import jax
import jax.numpy as jnp
from jax import lax
from jax.experimental import pallas as pl
from jax.experimental.pallas import tpu as pltpu
from jax.experimental.pallas import tpu_sc as plsc

NUM_EMB = 1000000
D = 64
BATCH = 4096
SEQ = 200
NC = 2
NS = 16
NW = NC * NS
BBLK = BATCH // NW
SP = 2
NSTEP = SEQ // SP
NBUF = 2
QROWS = SEQ * (D // 8) * NW * 8


def _embed_body(idxt_hbm, table_hbm, out_hbm, idx_v, rows_v, tp_v, qidx_v,
                gsems, ssems):
    wid = lax.axis_index("s") * NC + lax.axis_index("c")
    pltpu.sync_copy(idxt_hbm.at[:, pl.ds(wid * BBLK, BBLK)], idx_v)

    lane = jax.lax.iota(jnp.int32, 16)

    def fire_gather(step, b):
        for j in range(SP):
            pltpu.async_copy(
                table_hbm.at[idx_v.at[step * SP + j, pl.ds(0, BBLK)]],
                rows_v.at[b, pl.ds(j * BBLK, BBLK)],
                gsems[b],
            )

    def wait_gather(b):
        pltpu.make_async_copy(
            table_hbm.at[pl.ds(0, SP * BBLK)], rows_v.at[b], gsems[b]
        ).wait()

    def transpose_and_index(step, b):
        def cstep(c, carry):
            rbase = c * 16 + lane
            sl = c // (BBLK // 16)
            cl = c % (BBLK // 16)
            for d in range(D):
                vec = plsc.load_gather(
                    rows_v.at[b], [rbase, jnp.full((16,), d, jnp.int32)]
                )
                tp_v[b, sl * D + d, pl.ds(cl * 16, 16)] = vec
            return carry

        lax.fori_loop(0, SP * BBLK // 16, cstep, 0)

        for c in range(SP * D // 16):
            jv = c * 16 + lane
            slv = jv // D
            dv = jv - slv * D
            qv = ((step * SP + slv) * (2048 // 8) + (dv // 8) * 32 + wid) * 8 \
                + (dv - (dv // 8) * 8)
            qidx_v[b, pl.ds(c * 16, 16)] = qv

    def fire_store(b):
        pltpu.async_copy(
            tp_v.at[b], out_hbm.at[qidx_v.at[b, pl.ds(0, SP * D)]], ssems[b]
        )

    def wait_store(b):
        pltpu.make_async_copy(
            tp_v.at[b], out_hbm.at[pl.ds(0, SP * D)], ssems[b]
        ).wait()

    for b in range(NBUF):
        fire_gather(b, b)

    def group(g, carry):
        for b in range(NBUF):
            step = g * NBUF + b
            wait_gather(b)
            transpose_and_index(step, b)
            fire_store(b)
            wait_store(b)
            fire_gather(step + NBUF, b)
        return carry

    lax.fori_loop(0, NSTEP // NBUF - 1, group, 0)

    for b in range(NBUF):
        step = NSTEP - NBUF + b
        wait_gather(b)
        transpose_and_index(step, b)
        fire_store(b)
    for b in range(NBUF):
        wait_store(b)


@jax.jit
def _embed(idxt, embedding):
    mesh = plsc.VectorSubcoreMesh(
        core_axis_name="c", subcore_axis_name="s", num_cores=NC, num_subcores=NS
    )
    return pl.kernel(
        _embed_body,
        out_type=jax.ShapeDtypeStruct((QROWS, BBLK), jnp.float32),
        mesh=mesh,
        scratch_types=[
            pltpu.VMEM((SEQ, BBLK), jnp.int32),
            pltpu.VMEM((NBUF, SP * BBLK, D), jnp.float32),
            pltpu.VMEM((NBUF, SP * D, BBLK), jnp.float32),
            pltpu.VMEM((NBUF, SP * D), jnp.int32),
            [pltpu.SemaphoreType.DMA] * NBUF,
            [pltpu.SemaphoreType.DMA] * NBUF,
        ],
        compiler_params=pltpu.CompilerParams(
            use_tc_tiling_on_sc=False, needs_layout_passes=False
        ),
    )(idxt, embedding)


def kernel(inputs, embedding):
    idxt = jnp.transpose(inputs)
    out2 = _embed(idxt, embedding)
    out5 = out2.reshape(SEQ, D // 8, NW, 8, BBLK)
    return out5.transpose(2, 4, 0, 1, 3).reshape(BATCH, SEQ, D)


# speedup vs baseline: 1.3309x; 1.3255x over previous
import functools

import jax
import jax.numpy as jnp
from jax import lax
from jax.experimental import pallas as pl
from jax.experimental.pallas import tpu as pltpu
from jax.experimental.pallas import tpu_sc as plsc

NUM_EMB = 1000000
D = 64
BATCH = 4096
SEQ = 200
NC = 2
NS = 16
NW = NC * NS
BBLK = BATCH // NW
SP = 2
NSTEP = SEQ // SP
NBUF = 2
QROWS = SEQ * (D // 8) * NW * 8


def _embed_body(idxt_hbm, table_hbm, out_hbm, idx_v, rows_v, tp_v, qidx_v,
                gsems, ssems):
    wid = lax.axis_index("s") * NC + lax.axis_index("c")
    pltpu.sync_copy(idxt_hbm.at[:, pl.ds(wid * BBLK, BBLK)], idx_v)

    lane = jax.lax.iota(jnp.int32, 16)

    def fire_gather(step, b):
        for j in range(SP):
            pltpu.async_copy(
                table_hbm.at[idx_v.at[step * SP + j, pl.ds(0, BBLK)]],
                rows_v.at[b, pl.ds(j * BBLK, BBLK)],
                gsems[b],
            )

    def wait_gather(b):
        pltpu.make_async_copy(
            table_hbm.at[pl.ds(0, SP * BBLK)], rows_v.at[b], gsems[b]
        ).wait()

    def transpose_and_index(step, b):
        @plsc.parallel_loop(0, SP * BBLK // 16, unroll=2)
        def cstep(c):
            rbase = c * 16 + lane
            sl = c // (BBLK // 16)
            cl = c % (BBLK // 16)
            for d in range(D):
                vec = plsc.load_gather(
                    rows_v.at[b], [rbase, jnp.full((16,), d, jnp.int32)]
                )
                tp_v[b, sl * D + d, pl.ds(cl * 16, 16)] = vec

        for c in range(SP * D // 16):
            jv = c * 16 + lane
            slv = jv // D
            dv = jv - slv * D
            qv = ((step * SP + slv) * (2048 // 8) + (dv // 8) * 32 + wid) * 8 \
                + (dv - (dv // 8) * 8)
            qidx_v[b, pl.ds(c * 16, 16)] = qv

    def fire_store(b):
        pltpu.async_copy(
            tp_v.at[b], out_hbm.at[qidx_v.at[b, pl.ds(0, SP * D)]], ssems[b]
        )

    def wait_store(b):
        pltpu.make_async_copy(
            tp_v.at[b], out_hbm.at[pl.ds(0, SP * D)], ssems[b]
        ).wait()

    for b in range(NBUF):
        fire_gather(b, b)

    def group(g, carry):
        for b in range(NBUF):
            step = g * NBUF + b
            wait_gather(b)
            transpose_and_index(step, b)
            fire_store(b)
            wait_store(b)
            fire_gather(step + NBUF, b)
        return carry

    lax.fori_loop(0, NSTEP // NBUF - 1, group, 0)

    for b in range(NBUF):
        step = NSTEP - NBUF + b
        wait_gather(b)
        transpose_and_index(step, b)
        fire_store(b)
    for b in range(NBUF):
        wait_store(b)


@jax.jit
def _embed(idxt, embedding):
    mesh = plsc.VectorSubcoreMesh(
        core_axis_name="c", subcore_axis_name="s", num_cores=NC, num_subcores=NS
    )
    return pl.kernel(
        _embed_body,
        out_type=jax.ShapeDtypeStruct((QROWS, BBLK), jnp.float32),
        mesh=mesh,
        scratch_types=[
            pltpu.VMEM((SEQ, BBLK), jnp.int32),
            pltpu.VMEM((NBUF, SP * BBLK, D), jnp.float32),
            pltpu.VMEM((NBUF, SP * D, BBLK), jnp.float32),
            pltpu.VMEM((NBUF, SP * D), jnp.int32),
            [pltpu.SemaphoreType.DMA] * NBUF,
            [pltpu.SemaphoreType.DMA] * NBUF,
        ],
        compiler_params=pltpu.CompilerParams(
            use_tc_tiling_on_sc=False, needs_layout_passes=False
        ),
    )(idxt, embedding)


def kernel(inputs, embedding):
    idxt = jnp.transpose(inputs)
    out2 = _embed(idxt, embedding)
    out5 = out2.reshape(SEQ, D // 8, NW, 8, BBLK)
    return out5.transpose(2, 4, 0, 1, 3).reshape(BATCH, SEQ, D)


# speedup vs baseline: 1.3807x; 1.0374x over previous
import functools

import jax
import jax.numpy as jnp
from jax import lax
from jax.experimental import pallas as pl
from jax.experimental.pallas import tpu as pltpu
from jax.experimental.pallas import tpu_sc as plsc

NUM_EMB = 1000000
D = 64
BATCH = 4096
SEQ = 200
NC = 2
NS = 16
NW = NC * NS
BBLK = BATCH // NW
SP = 2
NSTEP = SEQ // SP
NBUF = 2
QROWS = SEQ * (D // 8) * NW * 8


def _embed_body(idxt_hbm, table_hbm, out_hbm, idx_v, rows_v, tp_v, qidx_v,
                gsems, ssems):
    wid = lax.axis_index("s") * NC + lax.axis_index("c")
    pltpu.sync_copy(idxt_hbm.at[:, pl.ds(wid * BBLK, BBLK)], idx_v)

    lane = jax.lax.iota(jnp.int32, 16)

    def fire_gather(step, b):
        for j in range(SP):
            pltpu.async_copy(
                table_hbm.at[idx_v.at[step * SP + j, pl.ds(0, BBLK)]],
                rows_v.at[b, pl.ds(j * BBLK, BBLK)],
                gsems[b],
            )

    def wait_gather(b):
        pltpu.make_async_copy(
            table_hbm.at[pl.ds(0, SP * BBLK)], rows_v.at[b], gsems[b]
        ).wait()

    def transpose_and_index(step, b):
        @plsc.parallel_loop(0, SP * BBLK, unroll=4)
        def rstep(r):
            sl = r // BBLK
            cl = r - sl * BBLK
            colv = jnp.full((16,), cl, jnp.int32)
            for dc in range(D // 16):
                vec = rows_v[b, r, pl.ds(dc * 16, 16)]
                plsc.store_scatter(
                    tp_v.at[b], [sl * D + dc * 16 + lane, colv], vec
                )

        for c in range(SP * D // 16):
            jv = c * 16 + lane
            slv = jv // D
            dv = jv - slv * D
            qv = ((step * SP + slv) * (2048 // 8) + (dv // 8) * 32 + wid) * 8 \
                + (dv - (dv // 8) * 8)
            qidx_v[b, pl.ds(c * 16, 16)] = qv

    def fire_store(b):
        pltpu.async_copy(
            tp_v.at[b], out_hbm.at[qidx_v.at[b, pl.ds(0, SP * D)]], ssems[b]
        )

    def wait_store(b):
        pltpu.make_async_copy(
            tp_v.at[b], out_hbm.at[pl.ds(0, SP * D)], ssems[b]
        ).wait()

    for b in range(NBUF):
        fire_gather(b, b)

    def group(g, carry):
        for b in range(NBUF):
            step = g * NBUF + b
            wait_gather(b)
            transpose_and_index(step, b)
            fire_store(b)
            wait_store(b)
            fire_gather(step + NBUF, b)
        return carry

    lax.fori_loop(0, NSTEP // NBUF - 1, group, 0)

    for b in range(NBUF):
        step = NSTEP - NBUF + b
        wait_gather(b)
        transpose_and_index(step, b)
        fire_store(b)
    for b in range(NBUF):
        wait_store(b)


@jax.jit
def _embed(idxt, embedding):
    mesh = plsc.VectorSubcoreMesh(
        core_axis_name="c", subcore_axis_name="s", num_cores=NC, num_subcores=NS
    )
    return pl.kernel(
        _embed_body,
        out_type=jax.ShapeDtypeStruct((QROWS, BBLK), jnp.float32),
        mesh=mesh,
        scratch_types=[
            pltpu.VMEM((SEQ, BBLK), jnp.int32),
            pltpu.VMEM((NBUF, SP * BBLK, D), jnp.float32),
            pltpu.VMEM((NBUF, SP * D, BBLK), jnp.float32),
            pltpu.VMEM((NBUF, SP * D), jnp.int32),
            [pltpu.SemaphoreType.DMA] * NBUF,
            [pltpu.SemaphoreType.DMA] * NBUF,
        ],
        compiler_params=pltpu.CompilerParams(
            use_tc_tiling_on_sc=False, needs_layout_passes=False
        ),
    )(idxt, embedding)


def kernel(inputs, embedding):
    idxt = jnp.transpose(inputs)
    out2 = _embed(idxt, embedding)
    out5 = out2.reshape(SEQ, D // 8, NW, 8, BBLK)
    return out5.transpose(2, 4, 0, 1, 3).reshape(BATCH, SEQ, D)


# speedup vs baseline: 2.1155x; 1.5322x over previous
import functools

import jax
import jax.numpy as jnp
from jax import lax
from jax.experimental import pallas as pl
from jax.experimental.pallas import tpu as pltpu
from jax.experimental.pallas import tpu_sc as plsc

NUM_EMB = 1000000
D = 64
BATCH = 4096
SEQ = 200
B_TOTAL = BATCH * SEQ
NC = 2
NS = 16
NW = NC * NS
BBLK = BATCH // NW
SP = 4
NSTEP = SEQ // SP
NBUF = 2
NGROUP = NSTEP // NBUF


def _embed_body(idxt_hbm, table_hbm, out_hbm, idx_v, rows_v, gsems, ssems):
    wid = lax.axis_index("s") * NC + lax.axis_index("c")
    b0 = wid * BBLK
    pltpu.sync_copy(idxt_hbm.at[:, pl.ds(b0, BBLK)], idx_v)

    def fire_gather(step, b):
        for j in range(SP):
            pltpu.async_copy(
                table_hbm.at[idx_v.at[step * SP + j, pl.ds(0, BBLK)]],
                rows_v.at[b, pl.ds(j * BBLK, BBLK)],
                gsems[b],
            )

    def wait_gather(b):
        pltpu.make_async_copy(
            table_hbm.at[pl.ds(0, SP * BBLK)], rows_v.at[b], gsems[b]
        ).wait()

    def fire_store(step, b):
        for j in range(SP):
            pltpu.async_copy(
                rows_v.at[b, pl.ds(j * BBLK, BBLK)],
                out_hbm.at[pl.ds((step * SP + j) * BATCH + b0, BBLK), pl.ds(0, D)],
                ssems[b],
            )

    def wait_store(b):
        pltpu.make_async_copy(
            rows_v.at[b], out_hbm.at[pl.ds(0, SP * BBLK), pl.ds(0, D)], ssems[b]
        ).wait()

    for b in range(NBUF):
        fire_gather(b, b)

    def group(g, carry):
        for b in range(NBUF):
            step = g * NBUF + b
            wait_gather(b)
            fire_store(step, b)
            wait_store(b)
            fire_gather(step + NBUF, b)
        return carry

    lax.fori_loop(0, NGROUP - 1, group, 0)

    for b in range(NBUF):
        step = (NGROUP - 1) * NBUF + b
        wait_gather(b)
        fire_store(step, b)
    for b in range(NBUF):
        wait_store(b)


@jax.jit
def _embed(idxt, embedding):
    mesh = plsc.VectorSubcoreMesh(
        core_axis_name="c", subcore_axis_name="s", num_cores=NC, num_subcores=NS
    )
    return pl.kernel(
        _embed_body,
        out_type=jax.ShapeDtypeStruct((B_TOTAL, 128), jnp.float32),
        mesh=mesh,
        scratch_types=[
            pltpu.VMEM((SEQ, BBLK), jnp.int32),
            pltpu.VMEM((NBUF, SP * BBLK, D), jnp.float32),
            [pltpu.SemaphoreType.DMA] * NBUF,
            [pltpu.SemaphoreType.DMA] * NBUF,
        ],
        compiler_params=pltpu.CompilerParams(use_tc_tiling_on_sc=False),
    )(idxt, embedding)


def kernel(inputs, embedding):
    idxt = jnp.transpose(inputs)
    out = _embed(idxt, embedding)
    return out[:, :D].reshape(SEQ, BATCH, D).transpose(1, 0, 2)
